# baseline (speedup 1.0000x reference)
import functools

import jax
import jax.numpy as jnp
from jax import lax
from jax.experimental import pallas as pl
from jax.experimental.pallas import tpu as pltpu


def _prep_kernel(x_ref, wk_ref, wq_ref, w0_ref, *, decay):
    xb = x_ref[...]
    wk = wk_ref[...]
    wq = wq_ref[...]
    cols = []
    for b in range(xb.shape[0]):
        kb = lax.dot_general(xb[b], wk, (((1,), (1,)), ((), ())),
                             preferred_element_type=jnp.float32)
        qb = lax.dot_general(xb[b], wq, (((1,), (1,)), ((), ())),
                             preferred_element_type=jnp.float32)
        cols.append(jnp.mean(kb * qb, axis=-1, keepdims=True))
    w0_ref[...] = jnp.concatenate(cols, axis=1) * decay


def _diffusion_kernel(c_ref, w0s_ref, x_ref, wv_ref, g_ref, bb_ref,
                      out_ref, cc_ref, y1s_ref, *, nb, bm, decay, eps):
    s = pl.program_id(0)

    @pl.when(s < nb)
    def _mv1():
        cb = c_ref[...]
        y1 = jnp.dot(cb, w0s_ref[...],
                     preferred_element_type=jnp.float32)
        y1s_ref[pl.ds(s * bm, bm), :] = y1 * decay
        cc_ref[...] = cb

    @pl.when(s >= nb)
    def _mv2_final():
        i = s - nb
        cb = c_ref[...]
        y2 = jnp.dot(cb, y1s_ref[...],
                     preferred_element_type=jnp.float32)
        rows = pl.ds(i * bm, bm)
        wtot = (w0s_ref[rows, :] + y1s_ref[rows, :]) * (1.0 / decay) + y2
        nb_, cin = x_ref.shape[0] * bm, x_ref.shape[2]
        xb = x_ref[...].reshape(nb_, cin)
        wv = wv_ref[...]
        g = g_ref[...]
        beta = bb_ref[...]
        v = lax.dot_general(xb, wv, (((1,), (1,)), ((), ())),
                            preferred_element_type=jnp.float32)
        wcol = jnp.concatenate(
            [wtot[:, b:b + 1] for b in range(x_ref.shape[0])],
            axis=0)
        ob = v * wcol
        mu = jnp.mean(ob, axis=-1, keepdims=True)
        var = jnp.mean((ob - mu) ** 2, axis=-1, keepdims=True)
        res = (ob - mu) / jnp.sqrt(var + eps) * g + beta
        out_ref[...] = res.reshape(x_ref.shape[0], bm, wv.shape[0])


def kernel(x, connection_matrix, Wk, Wq, Wv, gamma, beta):
    B, N, Cin = x.shape
    KD = Wk.shape[0]
    Cout = Wv.shape[0]
    decay = 0.7
    eps = 1e-5

    BM1 = 1000
    w0s = pl.pallas_call(
        functools.partial(_prep_kernel, decay=decay),
        grid=(N // BM1,),
        in_specs=[
            pl.BlockSpec((B, BM1, Cin), lambda i: (0, i, 0)),
            pl.BlockSpec((KD, Cin), lambda i: (0, 0)),
            pl.BlockSpec((KD, Cin), lambda i: (0, 0)),
        ],
        out_specs=pl.BlockSpec((BM1, B), lambda i: (i, 0)),
        out_shape=jax.ShapeDtypeStruct((N, B), jnp.float32),
    )(x, Wk, Wq)

    BM = 200
    NB = N // BM

    out, c_copy = pl.pallas_call(
        functools.partial(_diffusion_kernel, nb=NB, bm=BM, decay=decay,
                          eps=eps),
        grid=(2 * NB,),
        in_specs=[
            pl.BlockSpec((BM, N), lambda s: (lax.rem(s, NB), 0)),
            pl.BlockSpec((N, B), lambda s: (0, 0)),
            pl.BlockSpec((B, BM, Cin),
                         lambda s: (0, jnp.maximum(s - NB, 0), 0)),
            pl.BlockSpec((Cout, Cin), lambda s: (0, 0)),
            pl.BlockSpec((1, Cout), lambda s: (0, 0)),
            pl.BlockSpec((1, Cout), lambda s: (0, 0)),
        ],
        out_specs=[
            pl.BlockSpec((B, BM, Cout),
                         lambda s: (0, jnp.maximum(s - NB, 0), 0)),
            pl.BlockSpec((BM, N), lambda s: (jnp.minimum(s, NB - 1), 0)),
        ],
        out_shape=[
            jax.ShapeDtypeStruct((B, N, Cout), jnp.float32),
            jax.ShapeDtypeStruct((N, N), jnp.float32),
        ],
        scratch_shapes=[
            pltpu.VMEM((N, B), jnp.float32),
        ],
    )(connection_matrix, w0s, x, Wv,
      gamma.reshape(1, Cout), beta.reshape(1, Cout))

    return (out, c_copy)


# speedup vs baseline: 1.0192x; 1.0192x over previous
import functools

import jax
import jax.numpy as jnp
from jax import lax
from jax.experimental import pallas as pl
from jax.experimental.pallas import tpu as pltpu


def _prep_kernel(x_ref, wk_ref, wq_ref, w0_ref, *, decay):
    xb = x_ref[...]
    wk = wk_ref[...]
    wq = wq_ref[...]
    cols = []
    for b in range(xb.shape[0]):
        kb = lax.dot_general(xb[b], wk, (((1,), (1,)), ((), ())),
                             preferred_element_type=jnp.float32)
        qb = lax.dot_general(xb[b], wq, (((1,), (1,)), ((), ())),
                             preferred_element_type=jnp.float32)
        cols.append(jnp.mean(kb * qb, axis=-1, keepdims=True))
    w0_ref[...] = jnp.concatenate(cols, axis=1) * decay


def _diffusion_kernel(c_ref, w0s_ref, x_ref, wv_ref, g_ref, bb_ref,
                      out_ref, cc_ref, y1s_ref, *, nb, bm, decay, eps):
    s = pl.program_id(0)

    @pl.when(s < nb)
    def _mv1():
        cb = c_ref[...]
        y1 = jnp.dot(cb, w0s_ref[...],
                     preferred_element_type=jnp.float32)
        y1s_ref[pl.ds(s * bm, bm), :] = y1 * decay
        cc_ref[...] = cb

    @pl.when(s >= nb)
    def _mv2_final():
        i = s - nb
        cb = c_ref[...]
        y2 = jnp.dot(cb, y1s_ref[...],
                     preferred_element_type=jnp.float32)
        rows = pl.ds(i * bm, bm)
        wtot = (w0s_ref[rows, :] + y1s_ref[rows, :]) * (1.0 / decay) + y2
        xb = x_ref[...]
        wv = wv_ref[...]
        g = g_ref[...]
        beta = bb_ref[...]
        for b in range(xb.shape[0]):
            vb = lax.dot_general(xb[b], wv, (((1,), (1,)), ((), ())),
                                 preferred_element_type=jnp.float32)
            ob = vb * wtot[:, b:b + 1]
            mu = jnp.mean(ob, axis=-1, keepdims=True)
            var = jnp.mean((ob - mu) ** 2, axis=-1, keepdims=True)
            out_ref[b] = (ob - mu) / jnp.sqrt(var + eps) * g + beta


def kernel(x, connection_matrix, Wk, Wq, Wv, gamma, beta):
    B, N, Cin = x.shape
    KD = Wk.shape[0]
    Cout = Wv.shape[0]
    decay = 0.7
    eps = 1e-5

    BM1 = 1000
    w0s = pl.pallas_call(
        functools.partial(_prep_kernel, decay=decay),
        grid=(N // BM1,),
        in_specs=[
            pl.BlockSpec((B, BM1, Cin), lambda i: (0, i, 0)),
            pl.BlockSpec((KD, Cin), lambda i: (0, 0)),
            pl.BlockSpec((KD, Cin), lambda i: (0, 0)),
        ],
        out_specs=pl.BlockSpec((BM1, B), lambda i: (i, 0)),
        out_shape=jax.ShapeDtypeStruct((N, B), jnp.float32),
    )(x, Wk, Wq)

    BM = 200
    NB = N // BM

    out, c_copy = pl.pallas_call(
        functools.partial(_diffusion_kernel, nb=NB, bm=BM, decay=decay,
                          eps=eps),
        grid=(2 * NB,),
        in_specs=[
            pl.BlockSpec((BM, N), lambda s: (lax.rem(s, NB), 0)),
            pl.BlockSpec((N, B), lambda s: (0, 0)),
            pl.BlockSpec((B, BM, Cin),
                         lambda s: (0, jnp.maximum(s - NB, 0), 0)),
            pl.BlockSpec((Cout, Cin), lambda s: (0, 0)),
            pl.BlockSpec((1, Cout), lambda s: (0, 0)),
            pl.BlockSpec((1, Cout), lambda s: (0, 0)),
        ],
        out_specs=[
            pl.BlockSpec((B, BM, Cout),
                         lambda s: (0, jnp.maximum(s - NB, 0), 0)),
            pl.BlockSpec((BM, N), lambda s: (jnp.minimum(s, NB - 1), 0)),
        ],
        out_shape=[
            jax.ShapeDtypeStruct((B, N, Cout), jnp.float32),
            jax.ShapeDtypeStruct((N, N), jnp.float32),
        ],
        scratch_shapes=[
            pltpu.VMEM((N, B), jnp.float32),
        ],
    )(connection_matrix, w0s, x, Wv,
      gamma.reshape(1, Cout), beta.reshape(1, Cout))

    return (out, c_copy)


# speedup vs baseline: 1.0316x; 1.0122x over previous
import functools

import jax
import jax.numpy as jnp
from jax import lax
from jax.experimental import pallas as pl
from jax.experimental.pallas import tpu as pltpu


def _prep_kernel(x_ref, wk_ref, wq_ref, w0_ref, *, decay):
    xb = x_ref[...]
    wk = wk_ref[...]
    wq = wq_ref[...]
    cols = []
    for b in range(xb.shape[0]):
        kb = lax.dot_general(xb[b], wk, (((1,), (1,)), ((), ())),
                             preferred_element_type=jnp.float32)
        qb = lax.dot_general(xb[b], wq, (((1,), (1,)), ((), ())),
                             preferred_element_type=jnp.float32)
        cols.append(jnp.mean(kb * qb, axis=-1, keepdims=True))
    w0_ref[...] = jnp.concatenate(cols, axis=1) * decay


def _diffusion_kernel(c_ref, w0s_ref, x_ref, wv_ref, g_ref, bb_ref,
                      out_ref, cc_ref, y1s_ref, *, nb, bm, decay, eps):
    s = pl.program_id(0)

    @pl.when(s < nb)
    def _mv1():
        cb = c_ref[...]
        y1 = jnp.dot(cb, w0s_ref[...],
                     preferred_element_type=jnp.float32)
        y1s_ref[pl.ds(s * bm, bm), :] = y1 * decay
        cc_ref[...] = cb

    @pl.when(s >= nb)
    def _mv2_final():
        i = s - nb
        cb = c_ref[...]
        y2 = jnp.dot(cb, y1s_ref[...],
                     preferred_element_type=jnp.float32)
        rows = pl.ds(i * bm, bm)
        wtot = (w0s_ref[rows, :] + y1s_ref[rows, :]) * (1.0 / decay) + y2
        xb = x_ref[...]
        wv = wv_ref[...]
        g = g_ref[...]
        beta = bb_ref[...]
        for b in range(xb.shape[0]):
            vb = lax.dot_general(xb[b], wv, (((1,), (1,)), ((), ())),
                                 preferred_element_type=jnp.float32)
            ob = vb * wtot[:, b:b + 1]
            mu = jnp.mean(ob, axis=-1, keepdims=True)
            var = jnp.mean((ob - mu) ** 2, axis=-1, keepdims=True)
            out_ref[b] = (ob - mu) / jnp.sqrt(var + eps) * g + beta


def kernel(x, connection_matrix, Wk, Wq, Wv, gamma, beta):
    B, N, Cin = x.shape
    KD = Wk.shape[0]
    Cout = Wv.shape[0]
    decay = 0.7
    eps = 1e-5

    BM1 = 2000
    w0s = pl.pallas_call(
        functools.partial(_prep_kernel, decay=decay),
        grid=(N // BM1,),
        in_specs=[
            pl.BlockSpec((B, BM1, Cin), lambda i: (0, i, 0)),
            pl.BlockSpec((KD, Cin), lambda i: (0, 0)),
            pl.BlockSpec((KD, Cin), lambda i: (0, 0)),
        ],
        out_specs=pl.BlockSpec((BM1, B), lambda i: (i, 0)),
        out_shape=jax.ShapeDtypeStruct((N, B), jnp.float32),
    )(x, Wk, Wq)

    BM = 200
    NB = N // BM

    out, c_copy = pl.pallas_call(
        functools.partial(_diffusion_kernel, nb=NB, bm=BM, decay=decay,
                          eps=eps),
        grid=(2 * NB,),
        in_specs=[
            pl.BlockSpec((BM, N), lambda s: (lax.rem(s, NB), 0)),
            pl.BlockSpec((N, B), lambda s: (0, 0)),
            pl.BlockSpec((B, BM, Cin),
                         lambda s: (0, jnp.maximum(s - NB, 0), 0)),
            pl.BlockSpec((Cout, Cin), lambda s: (0, 0)),
            pl.BlockSpec((1, Cout), lambda s: (0, 0)),
            pl.BlockSpec((1, Cout), lambda s: (0, 0)),
        ],
        out_specs=[
            pl.BlockSpec((B, BM, Cout),
                         lambda s: (0, jnp.maximum(s - NB, 0), 0)),
            pl.BlockSpec((BM, N), lambda s: (jnp.minimum(s, NB - 1), 0)),
        ],
        out_shape=[
            jax.ShapeDtypeStruct((B, N, Cout), jnp.float32),
            jax.ShapeDtypeStruct((N, N), jnp.float32),
        ],
        scratch_shapes=[
            pltpu.VMEM((N, B), jnp.float32),
        ],
    )(connection_matrix, w0s, x, Wv,
      gamma.reshape(1, Cout), beta.reshape(1, Cout))

    return (out, c_copy)
